# baseline (speedup 1.0000x reference)
import functools

import jax
import jax.numpy as jnp
from jax import lax
from jax.experimental import pallas as pl
from jax.experimental.pallas import tpu as pltpu
from jax.experimental.pallas import tpu_sc as plsc

D_MODEL = 1024
D_FF = 4096
N_PATTERNS = 128
TOPK = 8
TOKEN_BLOCK = 512
NUM_WORKERS = 32
WCOLS = 64
LANES = 16
UNROLL = 4


def _scores_body(x_ref, r_ref, pata_ref, patb_ref, pb_ref, st_ref):
    xb = x_ref[...]
    rb = r_ref[...]
    psf = jax.lax.dot_general(pata_ref[...], xb, (((1,), (1,)), ((), ())),
                              preferred_element_type=jnp.float32)
    rsf = jax.lax.dot_general(patb_ref[...], rb, (((1,), (1,)), ((), ())),
                              preferred_element_type=jnp.float32)
    l0 = psf[N_PATTERNS, :] + rsf[N_PATTERNS, :] + pb_ref[0, 0]
    l1 = psf[N_PATTERNS + 1, :] + rsf[N_PATTERNS + 1, :] + pb_ref[0, 1]
    w0 = jax.nn.sigmoid(l0 - l1)[None, :]
    st_ref[...] = (w0 * psf[:N_PATTERNS, :]
                   + (1.0 - w0) * rsf[:N_PATTERNS, :])


def _routing_sc(scores_t):
    S = scores_t.shape[1]
    cols = 128
    n_workers = S // WCOLS
    groups = WCOLS // LANES
    mesh = plsc.VectorSubcoreMesh(core_axis_name="c", subcore_axis_name="s")

    @functools.partial(
        pl.kernel, mesh=mesh,
        out_type=jax.ShapeDtypeStruct((n_workers, N_PATTERNS, WCOLS),
                                      jnp.float32),
        scratch_types=[
            pltpu.VMEM((N_PATTERNS, cols), jnp.float32),
            pltpu.VMEM((N_PATTERNS, WCOLS), jnp.float32),
        ],
    )
    def k(st_hbm, out_hbm, sbuf, wbuf):
        c = lax.axis_index("c")
        s = lax.axis_index("s")
        strip = c * 8 + (s // 2)
        half = s % 2
        wid = c * 16 + s
        pltpu.sync_copy(st_hbm.at[:, pl.ds(strip * cols, cols)], sbuf)
        off = half * WCOLS
        neg = jnp.full((LANES,), -3.0e38, jnp.float32)
        for g in range(groups):
            sl = pl.ds(off + g * LANES, LANES)
            wsl = pl.ds(g * LANES, LANES)

            def insert(p4, t):
                t = list(t)
                for u in range(UNROLL):
                    b = sbuf[p4 * UNROLL + u, sl]
                    for j in range(TOPK):
                        hi = jnp.maximum(t[j], b)
                        b = jnp.minimum(t[j], b)
                        t[j] = hi
                return tuple(t)

            t = lax.fori_loop(0, N_PATTERNS // UNROLL, insert,
                              (neg,) * TOPK)
            denom = jnp.zeros((LANES,), jnp.float32)
            for j in range(TOPK):
                denom = denom + jnp.exp(t[j] - t[0])
            invd = 1.0 / denom
            thr = t[TOPK - 1]
            top = t[0]

            def emit(p4, carry):
                for u in range(UNROLL):
                    sv = sbuf[p4 * UNROLL + u, sl]
                    w = jnp.where(sv >= thr, jnp.exp(sv - top) * invd, 0.0)
                    wbuf[p4 * UNROLL + u, wsl] = w
                return carry

            lax.fori_loop(0, N_PATTERNS // UNROLL, emit, 0)
        pltpu.sync_copy(wbuf, out_hbm.at[wid])

    return k(scores_t)


def _ffn_body(x_ref, wm_ref, gates_ref, upw_ref, upb_ref, dww_ref,
              dwb_ref, out_ref):
    xb = x_ref[...]

    gates = gates_ref[...]
    pieces = []
    for j in range(TOKEN_BLOCK // WCOLS):
        pieces.append(jax.lax.dot_general(wm_ref[j], gates,
                                          (((0,), (0,)), ((), ())),
                                          preferred_element_type=jnp.float32))
    ffn_gate = jnp.concatenate(pieces, axis=0)

    h = jax.lax.dot_general(xb, upw_ref[...], (((1,), (1,)), ((), ())),
                            preferred_element_type=jnp.float32)
    h = h + upb_ref[...]
    h = h * jax.nn.sigmoid(ffn_gate)
    h = 0.5 * h * (1.0 + jax.lax.erf(h * 0.7071067811865476))
    out = jax.lax.dot_general(h, dww_ref[...], (((1,), (1,)), ((), ())),
                              preferred_element_type=jnp.float32)
    out_ref[...] = out + dwb_ref[...]


@functools.partial(jax.jit, static_argnames=())
def kernel(x, router_out, patterns, gates, path_w, path_b, up_w, up_b,
           down_w, down_b):
    B, S, _ = x.shape
    x2 = x.reshape(B * S, D_MODEL)
    r2 = router_out.reshape(B * S, D_MODEL)
    pad = jnp.zeros((126, D_MODEL), jnp.float32)
    pata = jnp.concatenate([patterns, path_w[:, :D_MODEL], pad], axis=0)
    patb = jnp.concatenate([patterns, path_w[:, D_MODEL:], pad], axis=0)
    pb2 = path_b.reshape(1, 2)
    upb2 = up_b.reshape(1, D_FF)
    dwb2 = down_b.reshape(1, D_MODEL)

    n_blocks = (B * S) // TOKEN_BLOCK
    full = lambda shape: pl.BlockSpec(shape, lambda i: (0,) * len(shape))

    scores_t = pl.pallas_call(
        _scores_body,
        grid=(n_blocks,),
        in_specs=[
            pl.BlockSpec((TOKEN_BLOCK, D_MODEL), lambda i: (i, 0)),
            pl.BlockSpec((TOKEN_BLOCK, D_MODEL), lambda i: (i, 0)),
            full((2 * N_PATTERNS, D_MODEL)),
            full((2 * N_PATTERNS, D_MODEL)),
            full((1, 2)),
        ],
        out_specs=pl.BlockSpec((N_PATTERNS, TOKEN_BLOCK), lambda i: (0, i)),
        out_shape=jax.ShapeDtypeStruct((N_PATTERNS, B * S), jnp.float32),
    )(x2, r2, pata, patb, pb2)

    routing_t = _routing_sc(scores_t)

    out = pl.pallas_call(
        _ffn_body,
        grid=(n_blocks,),
        in_specs=[
            pl.BlockSpec((TOKEN_BLOCK, D_MODEL), lambda i: (i, 0)),
            pl.BlockSpec((TOKEN_BLOCK // WCOLS, N_PATTERNS, WCOLS),
                         lambda i: (i, 0, 0)),
            full((N_PATTERNS, D_FF)),
            full((D_FF, D_MODEL)),
            full((1, D_FF)),
            full((D_MODEL, D_FF)),
            full((1, D_MODEL)),
        ],
        out_specs=pl.BlockSpec((TOKEN_BLOCK, D_MODEL), lambda i: (i, 0)),
        out_shape=jax.ShapeDtypeStruct((B * S, D_MODEL), jnp.float32),
    )(x2, routing_t, gates, up_w, upb2, down_w, dwb2)
    return out.reshape(B, S, D_MODEL)


# speedup vs baseline: 1.0528x; 1.0528x over previous
import functools

import jax
import jax.numpy as jnp
from jax import lax
from jax.experimental import pallas as pl
from jax.experimental.pallas import tpu as pltpu
from jax.experimental.pallas import tpu_sc as plsc

D_MODEL = 1024
D_FF = 4096
N_PATTERNS = 128
TOPK = 8
TOKEN_BLOCK = 512
NUM_WORKERS = 32
WCOLS = 64
LANES = 16
UNROLL = 4


def _scores_body(x_ref, r_ref, pata_ref, patb_ref, pb_ref, st_ref):
    xb = x_ref[...]
    rb = r_ref[...]
    psf = jax.lax.dot_general(pata_ref[...], xb, (((1,), (1,)), ((), ())),
                              preferred_element_type=jnp.float32)
    rsf = jax.lax.dot_general(patb_ref[...], rb, (((1,), (1,)), ((), ())),
                              preferred_element_type=jnp.float32)
    l0 = psf[N_PATTERNS, :] + rsf[N_PATTERNS, :] + pb_ref[0, 0]
    l1 = psf[N_PATTERNS + 1, :] + rsf[N_PATTERNS + 1, :] + pb_ref[0, 1]
    w0 = jax.nn.sigmoid(l0 - l1)[None, :]
    st_ref[...] = (w0 * psf[:N_PATTERNS, :]
                   + (1.0 - w0) * rsf[:N_PATTERNS, :])


def _routing_sc(scores_t):
    S = scores_t.shape[1]
    cols = 128
    per_core = (S // cols) // 2
    groups = cols // LANES
    mesh = plsc.VectorSubcoreMesh(core_axis_name="c", subcore_axis_name="s")

    @functools.partial(
        pl.kernel, mesh=mesh,
        out_type=jax.ShapeDtypeStruct((N_PATTERNS, S), jnp.float32),
        scratch_types=[
            pltpu.VMEM((N_PATTERNS, cols), jnp.float32),
            pltpu.VMEM((N_PATTERNS, cols), jnp.float32),
        ],
    )
    def k(st_hbm, out_hbm, sbuf, wbuf):
        c = lax.axis_index("c")
        s = lax.axis_index("s")

        @pl.when(s < per_core)
        def _work():
            base = (c * per_core + s) * cols
            pltpu.sync_copy(st_hbm.at[:, pl.ds(base, cols)], sbuf)
            _strips(sbuf, wbuf)
            pltpu.sync_copy(wbuf, out_hbm.at[:, pl.ds(base, cols)])

    def _strips(sbuf, wbuf):
        neg = jnp.full((LANES,), -3.0e38, jnp.float32)
        for g in range(groups):
            sl = pl.ds(g * LANES, LANES)
            wsl = sl

            def insert(p4, t):
                t = list(t)
                for u in range(UNROLL):
                    b = sbuf[p4 * UNROLL + u, sl]
                    for j in range(TOPK):
                        hi = jnp.maximum(t[j], b)
                        b = jnp.minimum(t[j], b)
                        t[j] = hi
                return tuple(t)

            t = lax.fori_loop(0, N_PATTERNS // UNROLL, insert,
                              (neg,) * TOPK)
            denom = jnp.zeros((LANES,), jnp.float32)
            for j in range(TOPK):
                denom = denom + jnp.exp(t[j] - t[0])
            invd = 1.0 / denom
            thr = t[TOPK - 1]
            top = t[0]

            def emit(p4, carry):
                for u in range(UNROLL):
                    sv = sbuf[p4 * UNROLL + u, sl]
                    w = jnp.where(sv >= thr, jnp.exp(sv - top) * invd, 0.0)
                    wbuf[p4 * UNROLL + u, wsl] = w
                return carry

            lax.fori_loop(0, N_PATTERNS // UNROLL, emit, 0)

    return k(scores_t)


def _ffn_body(x_ref, wm_ref, gates_ref, upw_ref, upb_ref, dww_ref,
              dwb_ref, out_ref):
    xb = x_ref[...]

    ffn_gate = jax.lax.dot_general(wm_ref[...], gates_ref[...],
                                   (((0,), (0,)), ((), ())),
                                   preferred_element_type=jnp.float32)

    h = jax.lax.dot_general(xb, upw_ref[...], (((1,), (1,)), ((), ())),
                            preferred_element_type=jnp.float32)
    h = h + upb_ref[...]
    h = h * jax.nn.sigmoid(ffn_gate)
    h = 0.5 * h * (1.0 + jax.lax.erf(h * 0.7071067811865476))
    out = jax.lax.dot_general(h, dww_ref[...], (((1,), (1,)), ((), ())),
                              preferred_element_type=jnp.float32)
    out_ref[...] = out + dwb_ref[...]


@functools.partial(jax.jit, static_argnames=())
def kernel(x, router_out, patterns, gates, path_w, path_b, up_w, up_b,
           down_w, down_b):
    B, S, _ = x.shape
    x2 = x.reshape(B * S, D_MODEL)
    r2 = router_out.reshape(B * S, D_MODEL)
    pad = jnp.zeros((126, D_MODEL), jnp.float32)
    pata = jnp.concatenate([patterns, path_w[:, :D_MODEL], pad], axis=0)
    patb = jnp.concatenate([patterns, path_w[:, D_MODEL:], pad], axis=0)
    pb2 = path_b.reshape(1, 2)
    upb2 = up_b.reshape(1, D_FF)
    dwb2 = down_b.reshape(1, D_MODEL)

    n_blocks = (B * S) // TOKEN_BLOCK
    full = lambda shape: pl.BlockSpec(shape, lambda i: (0,) * len(shape))

    scores_t = pl.pallas_call(
        _scores_body,
        grid=(n_blocks,),
        in_specs=[
            pl.BlockSpec((TOKEN_BLOCK, D_MODEL), lambda i: (i, 0)),
            pl.BlockSpec((TOKEN_BLOCK, D_MODEL), lambda i: (i, 0)),
            full((2 * N_PATTERNS, D_MODEL)),
            full((2 * N_PATTERNS, D_MODEL)),
            full((1, 2)),
        ],
        out_specs=pl.BlockSpec((N_PATTERNS, TOKEN_BLOCK), lambda i: (0, i)),
        out_shape=jax.ShapeDtypeStruct((N_PATTERNS, B * S), jnp.float32),
    )(x2, r2, pata, patb, pb2)

    routing_t = _routing_sc(scores_t)

    out = pl.pallas_call(
        _ffn_body,
        grid=(n_blocks,),
        in_specs=[
            pl.BlockSpec((TOKEN_BLOCK, D_MODEL), lambda i: (i, 0)),
            pl.BlockSpec((N_PATTERNS, TOKEN_BLOCK), lambda i: (0, i)),
            full((N_PATTERNS, D_FF)),
            full((D_FF, D_MODEL)),
            full((1, D_FF)),
            full((D_MODEL, D_FF)),
            full((1, D_MODEL)),
        ],
        out_specs=pl.BlockSpec((TOKEN_BLOCK, D_MODEL), lambda i: (i, 0)),
        out_shape=jax.ShapeDtypeStruct((B * S, D_MODEL), jnp.float32),
    )(x2, routing_t, gates, up_w, upb2, down_w, dwb2)
    return out.reshape(B, S, D_MODEL)


# speedup vs baseline: 1.0612x; 1.0080x over previous
import functools

import jax
import jax.numpy as jnp
from jax import lax
from jax.experimental import pallas as pl
from jax.experimental.pallas import tpu as pltpu
from jax.experimental.pallas import tpu_sc as plsc

D_MODEL = 1024
D_FF = 4096
N_PATTERNS = 128
TOPK = 8
TOKEN_BLOCK = 512
NUM_WORKERS = 32
WCOLS = 64
LANES = 16
UNROLL = 4


def _scores_body(x_ref, r_ref, pata_ref, patb_ref, pb_ref, st_ref):
    xb = x_ref[...]
    rb = r_ref[...]
    psf = jax.lax.dot_general(pata_ref[...], xb, (((1,), (1,)), ((), ())),
                              preferred_element_type=jnp.float32)
    rsf = jax.lax.dot_general(patb_ref[...], rb, (((1,), (1,)), ((), ())),
                              preferred_element_type=jnp.float32)
    l0 = psf[N_PATTERNS, :] + rsf[N_PATTERNS, :] + pb_ref[0, 0]
    l1 = psf[N_PATTERNS + 1, :] + rsf[N_PATTERNS + 1, :] + pb_ref[0, 1]
    w0 = jax.nn.sigmoid(l0 - l1)[None, :]
    st_ref[...] = (w0 * psf[:N_PATTERNS, :]
                   + (1.0 - w0) * rsf[:N_PATTERNS, :])


def _routing_sc(scores_t):
    S = scores_t.shape[1]
    cols = 128
    per_core = (S // cols) // 2
    groups = cols // LANES
    mesh = plsc.VectorSubcoreMesh(core_axis_name="c", subcore_axis_name="s")

    @functools.partial(
        pl.kernel, mesh=mesh,
        out_type=jax.ShapeDtypeStruct((N_PATTERNS, S), jnp.float32),
        scratch_types=[
            pltpu.VMEM((N_PATTERNS, cols), jnp.float32),
            pltpu.VMEM((N_PATTERNS, cols), jnp.float32),
        ],
    )
    def k(st_hbm, out_hbm, sbuf, wbuf):
        c = lax.axis_index("c")
        s = lax.axis_index("s")

        @pl.when(s < per_core)
        def _work():
            base = (c * per_core + s) * cols
            pltpu.sync_copy(st_hbm.at[:, pl.ds(base, cols)], sbuf)
            _strips(sbuf, wbuf)
            pltpu.sync_copy(wbuf, out_hbm.at[:, pl.ds(base, cols)])

    def _strips(sbuf, wbuf):
        neg = jnp.full((LANES,), -3.0e38, jnp.float32)
        npair = groups // 2
        for gp in range(npair):
            sls = (pl.ds(gp * LANES, LANES),
                   pl.ds((gp + npair) * LANES, LANES))

            def insert(p4, t):
                ta = list(t[:TOPK])
                tb = list(t[TOPK:])
                for u in range(UNROLL):
                    ba = sbuf[p4 * UNROLL + u, sls[0]]
                    bb = sbuf[p4 * UNROLL + u, sls[1]]
                    for j in range(TOPK):
                        ha = jnp.maximum(ta[j], ba)
                        hb = jnp.maximum(tb[j], bb)
                        ba = jnp.minimum(ta[j], ba)
                        bb = jnp.minimum(tb[j], bb)
                        ta[j] = ha
                        tb[j] = hb
                return tuple(ta) + tuple(tb)

            t = lax.fori_loop(0, N_PATTERNS // UNROLL, insert,
                              (neg,) * (2 * TOPK))
            parts = (t[:TOPK], t[TOPK:])
            stats = []
            for tg in parts:
                denom = jnp.zeros((LANES,), jnp.float32)
                for j in range(TOPK):
                    denom = denom + jnp.exp(tg[j] - tg[0])
                stats.append((tg[TOPK - 1], tg[0], 1.0 / denom))

            def emit(p4, carry):
                for u in range(UNROLL):
                    for sl, (thr, top, invd) in zip(sls, stats):
                        sv = sbuf[p4 * UNROLL + u, sl]
                        w = jnp.where(sv >= thr,
                                      jnp.exp(sv - top) * invd, 0.0)
                        wbuf[p4 * UNROLL + u, sl] = w
                return carry

            lax.fori_loop(0, N_PATTERNS // UNROLL, emit, 0)

    return k(scores_t)


def _ffn_body(x_ref, wm_ref, gates_ref, upw_ref, upb_ref, dww_ref,
              dwb_ref, out_ref):
    xb = x_ref[...]

    ffn_gate = jax.lax.dot_general(wm_ref[...], gates_ref[...],
                                   (((0,), (0,)), ((), ())),
                                   preferred_element_type=jnp.float32)

    h = jax.lax.dot_general(xb, upw_ref[...], (((1,), (1,)), ((), ())),
                            preferred_element_type=jnp.float32)
    h = h + upb_ref[...]
    h = h * jax.nn.sigmoid(ffn_gate)
    h = 0.5 * h * (1.0 + jax.lax.erf(h * 0.7071067811865476))
    out = jax.lax.dot_general(h, dww_ref[...], (((1,), (1,)), ((), ())),
                              preferred_element_type=jnp.float32)
    out_ref[...] = out + dwb_ref[...]


@functools.partial(jax.jit, static_argnames=())
def kernel(x, router_out, patterns, gates, path_w, path_b, up_w, up_b,
           down_w, down_b):
    B, S, _ = x.shape
    x2 = x.reshape(B * S, D_MODEL)
    r2 = router_out.reshape(B * S, D_MODEL)
    pad = jnp.zeros((126, D_MODEL), jnp.float32)
    pata = jnp.concatenate([patterns, path_w[:, :D_MODEL], pad], axis=0)
    patb = jnp.concatenate([patterns, path_w[:, D_MODEL:], pad], axis=0)
    pb2 = path_b.reshape(1, 2)
    upb2 = up_b.reshape(1, D_FF)
    dwb2 = down_b.reshape(1, D_MODEL)

    n_blocks = (B * S) // TOKEN_BLOCK
    full = lambda shape: pl.BlockSpec(shape, lambda i: (0,) * len(shape))

    scores_t = pl.pallas_call(
        _scores_body,
        grid=(n_blocks,),
        in_specs=[
            pl.BlockSpec((TOKEN_BLOCK, D_MODEL), lambda i: (i, 0)),
            pl.BlockSpec((TOKEN_BLOCK, D_MODEL), lambda i: (i, 0)),
            full((2 * N_PATTERNS, D_MODEL)),
            full((2 * N_PATTERNS, D_MODEL)),
            full((1, 2)),
        ],
        out_specs=pl.BlockSpec((N_PATTERNS, TOKEN_BLOCK), lambda i: (0, i)),
        out_shape=jax.ShapeDtypeStruct((N_PATTERNS, B * S), jnp.float32),
    )(x2, r2, pata, patb, pb2)

    routing_t = _routing_sc(scores_t)

    out = pl.pallas_call(
        _ffn_body,
        grid=(n_blocks,),
        in_specs=[
            pl.BlockSpec((TOKEN_BLOCK, D_MODEL), lambda i: (i, 0)),
            pl.BlockSpec((N_PATTERNS, TOKEN_BLOCK), lambda i: (0, i)),
            full((N_PATTERNS, D_FF)),
            full((D_FF, D_MODEL)),
            full((1, D_FF)),
            full((D_MODEL, D_FF)),
            full((1, D_MODEL)),
        ],
        out_specs=pl.BlockSpec((TOKEN_BLOCK, D_MODEL), lambda i: (i, 0)),
        out_shape=jax.ShapeDtypeStruct((B * S, D_MODEL), jnp.float32),
    )(x2, routing_t, gates, up_w, upb2, down_w, dwb2)
    return out.reshape(B, S, D_MODEL)
